# baseline (speedup 1.0000x reference)
import functools

import jax
import jax.numpy as jnp
from jax import lax
from jax.experimental import pallas as pl
from jax.experimental.pallas import tpu as pltpu
from jax.experimental.pallas import tpu_sc as plsc

NC, NS, L = 2, 16, 16
NW = NC * NS
NEG = -1e30


def _tc_pre(x, W1l, b1l, W1r, b1r, ea):
    N, F = x.shape
    D1 = W1l.shape[1]
    E = ea.shape[0]

    def body(x_ref, wl_ref, bl_ref, wr_ref, br_ref, ea_ref, xl_ref, xr_ref, m_ref):
        xv = x_ref[...]
        xl_ref[...] = (
            jnp.dot(xv, wl_ref[...], preferred_element_type=jnp.float32,
                    precision=lax.Precision.HIGHEST) + bl_ref[...][None, :])
        xr_ref[...] = (
            jnp.dot(xv, wr_ref[...], preferred_element_type=jnp.float32,
                    precision=lax.Precision.HIGHEST) + br_ref[...][None, :])
        m_ref[...] = jnp.sum(ea_ref[...], axis=0, keepdims=True) * (1.0 / E)

    return pl.pallas_call(
        body,
        out_shape=[
            jax.ShapeDtypeStruct((N, D1), jnp.float32),
            jax.ShapeDtypeStruct((N, D1), jnp.float32),
            jax.ShapeDtypeStruct((1, 1), jnp.float32),
        ],
    )(x, W1l, b1l, W1r, b1r, ea)


def _tc_mid(h_pre, b1, W2l, b2l, W2r, b2r):
    N, D1 = h_pre.shape
    D2 = W2l.shape[1]
    BR = 400
    G = N // BR

    def body(h_ref, b1_ref, wl_ref, bl_ref, wr_ref, br_ref, xl_ref, xr_ref):
        h1 = jnp.maximum(h_ref[...] + b1_ref[...][None, :], 0.0)
        xl_ref[...] = (
            jnp.dot(h1, wl_ref[...], preferred_element_type=jnp.float32,
                    precision=lax.Precision.HIGHEST) + bl_ref[...][None, :])
        xr_ref[...] = (
            jnp.dot(h1, wr_ref[...], preferred_element_type=jnp.float32,
                    precision=lax.Precision.HIGHEST) + br_ref[...][None, :])

    return pl.pallas_call(
        body,
        grid=(G,),
        in_specs=[
            pl.BlockSpec((BR, D1), lambda i: (i, 0)),
            pl.BlockSpec((D1,), lambda i: (0,)),
            pl.BlockSpec((D1, D2), lambda i: (0, 0)),
            pl.BlockSpec((D2,), lambda i: (0,)),
            pl.BlockSpec((D1, D2), lambda i: (0, 0)),
            pl.BlockSpec((D2,), lambda i: (0,)),
        ],
        out_specs=[
            pl.BlockSpec((BR, D2), lambda i: (i, 0)),
            pl.BlockSpec((BR, D2), lambda i: (i, 0)),
        ],
        out_shape=[
            jax.ShapeDtypeStruct((N, D2), jnp.float32),
            jax.ShapeDtypeStruct((N, D2), jnp.float32),
        ],
    )(h_pre, b1, W2l, b2l, W2r, b2r)


def _tc_post(pool_parts, alpha11, n_nodes, n_actions):
    _, D2 = pool_parts.shape

    def body(p_ref, a_ref, probs_ref, sig_ref):
        total = jnp.sum(p_ref[...], axis=0, keepdims=True) * (1.0 / n_nodes)
        col = lax.broadcasted_iota(jnp.int32, (1, D2), 1)
        masked = jnp.where(col < n_actions, total, NEG)
        m = jnp.max(masked, axis=1, keepdims=True)
        p = jnp.exp(masked - m)
        s = jnp.sum(p, axis=1, keepdims=True)
        probs_ref[...] = p / s
        sig_ref[...] = 1.0 / (1.0 + jnp.exp(-a_ref[...]))

    return pl.pallas_call(
        body,
        out_shape=[
            jax.ShapeDtypeStruct((1, D2), jnp.float32),
            jax.ShapeDtypeStruct((1, 1), jnp.float32),
        ],
    )(pool_parts, alpha11)


def _sc_layer(xl, xr, src, dst, eav, wev, attv, *, heads, chan, n_nodes,
              pooled, bias=None):
    N, D = xl.shape
    NE = src.shape[0]
    RPT = -(-(-(-n_nodes // NW)) // 8) * 8
    NPAD = RPT * NW
    NCH = -(-NE // L)
    NCHO = (RPT + L) // L
    LGS = NE + 2 * L
    D16 = D // L
    C16 = chan // L

    mesh = plsc.VectorSubcoreMesh(core_axis_name="c", subcore_axis_name="s")

    if pooled:
        out_shape = jax.ShapeDtypeStruct((NW, D), jnp.float32)
    else:
        out_shape = jax.ShapeDtypeStruct((NPAD, D), jnp.float32)

    scratch = [
        pltpu.VMEM((NE + L,), jnp.int32),
        pltpu.VMEM((NE + L,), jnp.int32),
        pltpu.VMEM((NE + L,), jnp.float32),
        pltpu.VMEM((NE + L,), jnp.int32),
        pltpu.VMEM((NE + L,), jnp.int32),
        pltpu.VMEM((NE + L,), jnp.int32),
        pltpu.VMEM((NE + L,), jnp.float32),
        pltpu.VMEM((heads * LGS,), jnp.float32),
        pltpu.VMEM((heads * LGS,), jnp.float32),
        pltpu.VMEM((NE + 4 * L,), jnp.int32),
        pltpu.VMEM((NCHO * L + L,), jnp.int32),
        pltpu.VMEM((NCHO * L + L,), jnp.int32),
        pltpu.VMEM((NCHO * L + L,), jnp.int32),
        pltpu.VMEM((D,), jnp.float32),
        pltpu.VMEM((D,), jnp.float32),
        pltpu.VMEM((D,), jnp.float32),
        pltpu.VMEM((L,), jnp.int32),
        pltpu.VMEM((L,), jnp.int32),
        pltpu.VMEM((L, D), jnp.float32),
        pltpu.VMEM((L, D), jnp.float32),
        pltpu.VMEM((D,), jnp.float32),
        pltpu.VMEM((D,), jnp.float32),
        (pltpu.VMEM((1, D), jnp.float32) if pooled
         else pltpu.VMEM((RPT, D), jnp.float32)),
        pltpu.SemaphoreType.DMA,
        pltpu.SemaphoreType.DMA,
    ]

    @functools.partial(
        pl.kernel,
        mesh=mesh,
        compiler_params=pltpu.CompilerParams(needs_layout_passes=False),
        out_type=out_shape,
        scratch_types=scratch,
    )
    def body(xl_hbm, xr_hbm, src_hbm, dst_hbm, ea_hbm, we_hbm, att_hbm, b_hbm,
             out_hbm, src_v, dst_v, ea_v, own_v, src_own, dst_own, ea_own,
             lg_v, w_v, sorted_v, deg_v, off_v, cur_v, we_v, att_v, b_v,
             gidx_a, gidx_b, rows_a, rows_b, acc_v, pool_v, outbuf_v,
             sem_a, sem_b):
        wid = lax.axis_index("s") * NC + lax.axis_index("c")
        lo = wid * RPT
        iota = lax.iota(jnp.int32, L)
        lane0 = iota == 0

        pltpu.sync_copy(src_hbm, src_v.at[pl.ds(0, NE)])
        pltpu.sync_copy(dst_hbm, dst_v.at[pl.ds(0, NE)])
        pltpu.sync_copy(ea_hbm, ea_v.at[pl.ds(0, NE)])
        pltpu.sync_copy(we_hbm, we_v)
        pltpu.sync_copy(att_hbm, att_v)
        pltpu.sync_copy(b_hbm, b_v)
        src_v[pl.ds(NE, L)] = jnp.zeros((L,), jnp.int32)
        dst_v[pl.ds(NE, L)] = jnp.full((L,), -1, jnp.int32)
        ea_v[pl.ds(NE, L)] = jnp.zeros((L,), jnp.float32)

        def cbody(c, cnt):
            dv = dst_v[pl.ds(c * L, L)]
            m = (dv >= lo) & (dv < lo + RPT)
            mi = m.astype(jnp.int32)
            incl = plsc.cumsum(mi)
            pos = cnt + incl - mi
            plsc.store_scatter(own_v, [pos], iota + c * L, mask=m)
            return cnt + incl[L - 1]

        cnt = lax.fori_loop(0, NCH, cbody, jnp.int32(0))
        own_v[pl.ds(cnt, L)] = jnp.full((L,), NE, jnp.int32)
        nchunks = lax.shift_right_logical(cnt + (L - 1), 4)

        def abody(c, carry):
            ids = own_v[pl.ds(c * L, L)]
            src_own[pl.ds(c * L, L)] = plsc.load_gather(src_v, [ids])
            dst_own[pl.ds(c * L, L)] = plsc.load_gather(dst_v, [ids])
            ea_own[pl.ds(c * L, L)] = plsc.load_gather(ea_v, [ids])
            return carry

        lax.fori_loop(0, nchunks, abody, 0)
        src_own[pl.ds(NE, L)] = jnp.zeros((L,), jnp.int32)
        dst_own[pl.ds(NE, L)] = jnp.full((L,), -1, jnp.int32)

        def zd(c, carry):
            deg_v[pl.ds(c * L, L)] = jnp.zeros((L,), jnp.int32)
            return carry

        lax.fori_loop(0, NCHO + 1, zd, 0)

        def dcount(j, carry):
            d = dst_own[pl.ds(j, L)][0] - lo
            cdeg = deg_v[pl.ds(d, L)][0]
            plsc.store_scatter(deg_v, [jnp.full((L,), d, jnp.int32)],
                               jnp.full((L,), cdeg + 1, jnp.int32), mask=lane0)
            return carry

        lax.fori_loop(0, cnt, dcount, 0)

        def obody(c, carry_off):
            v = deg_v[pl.ds(c * L, L)]
            incl = plsc.cumsum(v)
            off_v[pl.ds(c * L, L)] = carry_off + incl - v
            cur_v[pl.ds(c * L, L)] = carry_off + incl - v
            return carry_off + incl[L - 1]

        lax.fori_loop(0, NCHO, obody, jnp.int32(0))

        def sfill(j, carry):
            d = dst_own[pl.ds(j, L)][0] - lo
            p = cur_v[pl.ds(d, L)][0]
            plsc.store_scatter(sorted_v, [jnp.full((L,), p, jnp.int32)],
                               jnp.full((L,), j, jnp.int32), mask=lane0)
            plsc.store_scatter(cur_v, [jnp.full((L,), d, jnp.int32)],
                               jnp.full((L,), p + 1, jnp.int32), mask=lane0)
            return carry

        lax.fori_loop(0, cnt, sfill, 0)
        sorted_v[pl.ds(cnt, L)] = jnp.full((L,), NE, jnp.int32)
        sorted_v[pl.ds(cnt + L, L)] = jnp.full((L,), NE, jnp.int32)
        sorted_v[pl.ds(cnt + 2 * L, L)] = jnp.full((L,), NE, jnp.int32)
        sorted_v[pl.ds(cnt + 3 * L, L)] = jnp.full((L,), NE, jnp.int32)

        def gbody(g, carry):
            jpos = sorted_v[pl.ds(g * L, L)]
            sv = plsc.load_gather(src_own, [jpos])
            dv = plsc.load_gather(dst_own, [jpos])
            av = plsc.load_gather(ea_own, [jpos])
            gidx_a[...] = sv
            gidx_b[...] = jnp.maximum(dv, 0)
            cp_a = pltpu.async_copy(xl_hbm.at[gidx_a], rows_a, sem_a)
            cp_b = pltpu.async_copy(xr_hbm.at[gidx_b], rows_b, sem_b)
            cp_a.wait()
            cp_b.wait()
            a_s = [av[t] for t in range(L)]
            for h in range(heads):
                def kb(k, accs):
                    base = h * chan + k * L
                    wek = we_v[pl.ds(base, L)]
                    attk = att_v[pl.ds(base, L)]
                    out = []
                    for t in range(L):
                        v = (rows_a[t, pl.ds(base, L)] + rows_b[t, pl.ds(base, L)]
                             + a_s[t] * wek)
                        z = jnp.where(v >= 0.0, v, 0.2 * v)
                        out.append(accs[t] + z * attk)
                    return tuple(out)

                accs = lax.fori_loop(
                    0, C16, kb,
                    tuple(jnp.zeros((L,), jnp.float32) for _ in range(L)))
                for t in range(L):
                    plsc.store_scatter(
                        lg_v, [jnp.full((L,), h * LGS + g * L + t, jnp.int32)],
                        jnp.full((L,), jnp.sum(accs[t]), jnp.float32),
                        mask=lane0)
            return carry

        lax.fori_loop(0, nchunks, gbody, 0)

        def nbody(i, carry):
            s0 = off_v[pl.ds(i, L)][0]
            s1 = off_v[pl.ds(i + 1, L)][0]
            c0 = lax.shift_right_logical(s0, 4)
            c1 = lax.shift_right_logical(s1 + (L - 1), 4)

            def pa(c, mxs):
                idx = iota + c * L
                m = (idx >= s0) & (idx < s1)
                return tuple(
                    jnp.maximum(mxs[h],
                                jnp.where(m, lg_v[pl.ds(h * LGS + c * L, L)], NEG))
                    for h in range(heads))

            mxs = lax.fori_loop(
                c0, c1, pa,
                tuple(jnp.full((L,), NEG, jnp.float32) for _ in range(heads)))
            amax = [jnp.max(mxs[h]) for h in range(heads)]
            amax = [jnp.where(a < -1e29, 0.0, a) for a in amax]

            def pb(c, dsums):
                idx = iota + c * L
                m = (idx >= s0) & (idx < s1)
                return tuple(
                    dsums[h] + jnp.where(
                        m, jnp.exp(lg_v[pl.ds(h * LGS + c * L, L)] - amax[h]),
                        0.0)
                    for h in range(heads))

            dsums = lax.fori_loop(
                c0, c1, pb,
                tuple(jnp.zeros((L,), jnp.float32) for _ in range(heads)))
            winv = [1.0 / jnp.full((L,), jnp.sum(dsums[h]) + 1e-16, jnp.float32)
                    for h in range(heads)]

            def pw(c, carry2):
                idx = iota + c * L
                m = (idx >= s0) & (idx < s1)
                for h in range(heads):
                    wv = (jnp.exp(lg_v[pl.ds(h * LGS + c * L, L)] - amax[h])
                          * winv[h])
                    plsc.store_scatter(w_v, [h * LGS + idx], wv, mask=m)
                return carry2

            lax.fori_loop(c0, c1, pw, 0)
            return carry

        lax.fori_loop(0, RPT, nbody, 0)

        def prep(c, carry):
            jpos = sorted_v[pl.ds(c * L, L)]
            own_v[pl.ds(c * L, L)] = plsc.load_gather(src_own, [jpos])
            return carry

        lax.fori_loop(0, nchunks + 3, prep, 0)
        npairs = jnp.maximum(
            lax.shift_right_logical(nchunks + 1, 1), jnp.int32(1))

        def issue(g, buf, gidx, sem):
            gidx[...] = own_v[pl.ds(g * L, L)]
            pltpu.async_copy(xl_hbm.at[gidx], buf, sem)

        def wait_for(buf, gidx, sem):
            pltpu.make_async_copy(xl_hbm.at[gidx], buf, sem).wait()

        if pooled:
            def zp(k, carry):
                pool_v[pl.ds(k * L, L)] = jnp.zeros((L,), jnp.float32)
                acc_v[pl.ds(k * L, L)] = jnp.zeros((L,), jnp.float32)
                return carry
            lax.fori_loop(0, D16, zp, 0)

            def flush(nid):
                def fk(k, carry):
                    kb = k * L
                    plsc.addupdate(
                        pool_v.at[pl.ds(kb, L)],
                        jnp.maximum(acc_v[pl.ds(kb, L)] + b_v[pl.ds(kb, L)],
                                    0.0))
                    acc_v[pl.ds(kb, L)] = jnp.zeros((L,), jnp.float32)
                    return carry
                lax.fori_loop(0, D16, fk, 0)

            def process(g, rows, cur):
                jpos = sorted_v[pl.ds(g * L, L)]
                dstv = plsc.load_gather(dst_own, [jpos])
                for t in range(L):
                    d_t = dstv[t]

                    @pl.when((d_t >= 0) & (d_t != cur) & (cur >= 0))
                    def _():
                        flush(cur)

                    cur = jnp.where(d_t >= 0, d_t, cur)

                    @pl.when(d_t >= 0)
                    def _():
                        w_t = w_v[pl.ds(g * L + t, L)][0]

                        def kb2(k, carry2):
                            kb = k * L
                            plsc.addupdate(acc_v.at[pl.ds(kb, L)],
                                           w_t * rows[t, pl.ds(kb, L)])
                            return carry2

                        lax.fori_loop(0, D16, kb2, 0)
                return cur

            issue(0, rows_a, gidx_a, sem_a)

            def pair(p, cur):
                g0 = 2 * p
                issue(g0 + 1, rows_b, gidx_b, sem_b)
                wait_for(rows_a, gidx_a, sem_a)
                cur = process(g0, rows_a, cur)
                issue(g0 + 2, rows_a, gidx_a, sem_a)
                wait_for(rows_b, gidx_b, sem_b)
                cur = process(g0 + 1, rows_b, cur)
                return cur

            cur = lax.fori_loop(0, npairs, pair, jnp.int32(-1))
            wait_for(rows_a, gidx_a, sem_a)

            @pl.when(cur >= 0)
            def _():
                flush(cur)

            pltpu.sync_copy(pool_v, out_hbm.at[wid])
        else:
            def zo(i, carry):
                def zk(k, carry2):
                    outbuf_v[i, pl.ds(k * L, L)] = jnp.zeros((L,), jnp.float32)
                    return carry2
                lax.fori_loop(0, D16, zk, 0)
                return carry

            lax.fori_loop(0, RPT, zo, 0)

            def process(g, rows):
                jpos = sorted_v[pl.ds(g * L, L)]
                dstv = plsc.load_gather(dst_own, [jpos])
                for t in range(L):
                    d_t = dstv[t]

                    @pl.when(d_t >= 0)
                    def _():
                        dl = d_t - lo
                        for h in range(heads):
                            w_t = w_v[pl.ds(h * LGS + g * L + t, L)][0]

                            def kb2(k, carry2):
                                base = h * chan + k * L
                                plsc.addupdate(
                                    outbuf_v.at[dl, pl.ds(base, L)],
                                    w_t * rows[t, pl.ds(base, L)])
                                return carry2

                            lax.fori_loop(0, C16, kb2, 0)

            issue(0, rows_a, gidx_a, sem_a)

            def pair(p, carry):
                g0 = 2 * p
                issue(g0 + 1, rows_b, gidx_b, sem_b)
                wait_for(rows_a, gidx_a, sem_a)
                process(g0, rows_a)
                issue(g0 + 2, rows_a, gidx_a, sem_a)
                wait_for(rows_b, gidx_b, sem_b)
                process(g0 + 1, rows_b)
                return carry

            lax.fori_loop(0, npairs, pair, 0)
            wait_for(rows_a, gidx_a, sem_a)
            pltpu.sync_copy(outbuf_v, out_hbm.at[pl.ds(lo, RPT)])

    return body(xl, xr, src, dst, eav, wev, attv,
                bias if bias is not None else jnp.zeros((D,), jnp.float32))


def kernel(x, edge_index, edge_attr, g1_Wl, g1_bl, g1_Wr, g1_br, g1_We, g1_att,
           g1_b, g2_Wl, g2_bl, g2_Wr, g2_br, g2_We, g2_att, g2_b, alpha):
    N, F = x.shape
    E = edge_index.shape[1]
    NE = E + N
    H1, C1 = g1_att.shape
    D1 = H1 * C1
    A = E + 1
    D2 = -(-A // 128) * 128

    loop = jnp.arange(N, dtype=edge_index.dtype)
    src = jnp.concatenate([edge_index[0], loop])
    dst = jnp.concatenate([edge_index[1], loop])

    xl1, xr1, mean_ea = _tc_pre(x, g1_Wl, g1_bl, g1_Wr, g1_br, edge_attr)
    eav = jnp.concatenate(
        [edge_attr[:, 0], jnp.broadcast_to(mean_ea[0, 0], (N,))])

    h_pre = _sc_layer(
        xl1, xr1, src, dst, eav, g1_We[0], g1_att.reshape(D1),
        heads=H1, chan=C1, n_nodes=N, pooled=False)

    pad = D2 - A
    W2l = jnp.pad(g2_Wl, ((0, 0), (0, pad)))
    W2r = jnp.pad(g2_Wr, ((0, 0), (0, pad)))
    b2l = jnp.pad(g2_bl, (0, pad))
    b2r = jnp.pad(g2_br, (0, pad))
    we2 = jnp.pad(g2_We[0], (0, pad))
    att2 = jnp.pad(g2_att.reshape(A), (0, pad))
    b2 = jnp.pad(g2_b, (0, pad))

    xl2, xr2 = _tc_mid(h_pre[:N], g1_b, W2l, b2l, W2r, b2r)

    pool_parts = _sc_layer(
        xl2, xr2, src, dst, eav, we2, att2,
        heads=1, chan=D2, n_nodes=N, pooled=True, bias=b2)

    probs_full, sig11 = _tc_post(pool_parts, alpha.reshape(1, 1), N, A)
    return probs_full[:, :A], sig11.reshape(())


# speedup vs baseline: 1.0917x; 1.0917x over previous
import functools

import jax
import jax.numpy as jnp
from jax import lax
from jax.experimental import pallas as pl
from jax.experimental.pallas import tpu as pltpu
from jax.experimental.pallas import tpu_sc as plsc

NC, NS, L = 2, 16, 16
NW = NC * NS
NEG = -1e30


def _tc_pre(x, W1l, b1l, W1r, b1r, ea):
    N, F = x.shape
    D1 = W1l.shape[1]
    E = ea.shape[0]

    def body(x_ref, wl_ref, bl_ref, wr_ref, br_ref, ea_ref, xl_ref, xr_ref, m_ref):
        xv = x_ref[...]
        xl_ref[...] = (
            jnp.dot(xv, wl_ref[...], preferred_element_type=jnp.float32,
                    precision=lax.Precision.HIGHEST) + bl_ref[...][None, :])
        xr_ref[...] = (
            jnp.dot(xv, wr_ref[...], preferred_element_type=jnp.float32,
                    precision=lax.Precision.HIGHEST) + br_ref[...][None, :])
        m_ref[...] = jnp.sum(ea_ref[...], axis=0, keepdims=True) * (1.0 / E)

    return pl.pallas_call(
        body,
        out_shape=[
            jax.ShapeDtypeStruct((N, D1), jnp.float32),
            jax.ShapeDtypeStruct((N, D1), jnp.float32),
            jax.ShapeDtypeStruct((1, 1), jnp.float32),
        ],
    )(x, W1l, b1l, W1r, b1r, ea)


def _tc_mid(h_pre, b1, W2l, b2l, W2r, b2r):
    N, D1 = h_pre.shape
    D2 = W2l.shape[1]
    BR = 400
    G = N // BR

    def body(h_ref, b1_ref, wl_ref, bl_ref, wr_ref, br_ref, xl_ref, xr_ref):
        h1 = jnp.maximum(h_ref[...] + b1_ref[...][None, :], 0.0)
        xl_ref[...] = (
            jnp.dot(h1, wl_ref[...], preferred_element_type=jnp.float32,
                    precision=lax.Precision.HIGHEST) + bl_ref[...][None, :])
        xr_ref[...] = (
            jnp.dot(h1, wr_ref[...], preferred_element_type=jnp.float32,
                    precision=lax.Precision.HIGHEST) + br_ref[...][None, :])

    return pl.pallas_call(
        body,
        grid=(G,),
        in_specs=[
            pl.BlockSpec((BR, D1), lambda i: (i, 0)),
            pl.BlockSpec((D1,), lambda i: (0,)),
            pl.BlockSpec((D1, D2), lambda i: (0, 0)),
            pl.BlockSpec((D2,), lambda i: (0,)),
            pl.BlockSpec((D1, D2), lambda i: (0, 0)),
            pl.BlockSpec((D2,), lambda i: (0,)),
        ],
        out_specs=[
            pl.BlockSpec((BR, D2), lambda i: (i, 0)),
            pl.BlockSpec((BR, D2), lambda i: (i, 0)),
        ],
        out_shape=[
            jax.ShapeDtypeStruct((N, D2), jnp.float32),
            jax.ShapeDtypeStruct((N, D2), jnp.float32),
        ],
    )(h_pre, b1, W2l, b2l, W2r, b2r)


def _tc_post(pool_parts, alpha11, n_nodes, n_actions):
    _, D2 = pool_parts.shape

    def body(p_ref, a_ref, probs_ref, sig_ref):
        total = jnp.sum(p_ref[...], axis=0, keepdims=True) * (1.0 / n_nodes)
        col = lax.broadcasted_iota(jnp.int32, (1, D2), 1)
        masked = jnp.where(col < n_actions, total, NEG)
        m = jnp.max(masked, axis=1, keepdims=True)
        p = jnp.exp(masked - m)
        s = jnp.sum(p, axis=1, keepdims=True)
        probs_ref[...] = p / s
        sig_ref[...] = 1.0 / (1.0 + jnp.exp(-a_ref[...]))

    return pl.pallas_call(
        body,
        out_shape=[
            jax.ShapeDtypeStruct((1, D2), jnp.float32),
            jax.ShapeDtypeStruct((1, 1), jnp.float32),
        ],
    )(pool_parts, alpha11)


def _sc_layer(xl, xr, src, dst, eav, wev, attv, *, heads, chan, n_nodes,
              pooled, bias=None):
    N, D = xl.shape
    NE = src.shape[0]
    RPT = -(-(-(-n_nodes // NW)) // 8) * 8
    NPAD = RPT * NW
    NCH = -(-NE // L)
    NCHO = (RPT + L) // L
    LGS = NE + 2 * L
    D16 = D // L
    C16 = chan // L

    mesh = plsc.VectorSubcoreMesh(core_axis_name="c", subcore_axis_name="s")

    if pooled:
        out_shape = jax.ShapeDtypeStruct((NW, D), jnp.float32)
    else:
        out_shape = jax.ShapeDtypeStruct((NPAD, D), jnp.float32)

    scratch = [
        pltpu.VMEM((NE + L,), jnp.int32),
        pltpu.VMEM((NE + L,), jnp.int32),
        pltpu.VMEM((NE + L,), jnp.float32),
        pltpu.VMEM((NE + L,), jnp.int32),
        pltpu.VMEM((NE + L,), jnp.int32),
        pltpu.VMEM((NE + L,), jnp.int32),
        pltpu.VMEM((NE + L,), jnp.float32),
        pltpu.VMEM((heads * LGS,), jnp.float32),
        pltpu.VMEM((heads * LGS,), jnp.float32),
        pltpu.VMEM((NE + 4 * L,), jnp.int32),
        pltpu.VMEM((NCHO * L + L,), jnp.int32),
        pltpu.VMEM((NCHO * L + L,), jnp.int32),
        pltpu.VMEM((NCHO * L + L,), jnp.int32),
        pltpu.VMEM((D,), jnp.float32),
        pltpu.VMEM((D,), jnp.float32),
        pltpu.VMEM((D,), jnp.float32),
        pltpu.VMEM((L,), jnp.int32),
        pltpu.VMEM((L,), jnp.int32),
        pltpu.VMEM((L, D), jnp.float32),
        pltpu.VMEM((L, D), jnp.float32),
        pltpu.VMEM((D,), jnp.float32),
        pltpu.VMEM((D,), jnp.float32),
        (pltpu.VMEM((1, D), jnp.float32) if pooled
         else pltpu.VMEM((RPT, D), jnp.float32)),
        pltpu.SemaphoreType.DMA,
        pltpu.SemaphoreType.DMA,
    ]

    @functools.partial(
        pl.kernel,
        mesh=mesh,
        compiler_params=pltpu.CompilerParams(needs_layout_passes=False),
        out_type=out_shape,
        scratch_types=scratch,
    )
    def body(xl_hbm, xr_hbm, src_hbm, dst_hbm, ea_hbm, we_hbm, att_hbm, b_hbm,
             out_hbm, src_v, dst_v, ea_v, own_v, src_own, dst_own, ea_own,
             lg_v, w_v, sorted_v, deg_v, off_v, cur_v, we_v, att_v, b_v,
             gidx_a, gidx_b, rows_a, rows_b, acc_v, pool_v, outbuf_v,
             sem_a, sem_b):
        wid = lax.axis_index("s") * NC + lax.axis_index("c")
        lo = wid * RPT
        iota = lax.iota(jnp.int32, L)
        lane0 = iota == 0

        pltpu.sync_copy(src_hbm, src_v.at[pl.ds(0, NE)])
        pltpu.sync_copy(dst_hbm, dst_v.at[pl.ds(0, NE)])
        pltpu.sync_copy(ea_hbm, ea_v.at[pl.ds(0, NE)])
        pltpu.sync_copy(we_hbm, we_v)
        pltpu.sync_copy(att_hbm, att_v)
        pltpu.sync_copy(b_hbm, b_v)
        src_v[pl.ds(NE, L)] = jnp.zeros((L,), jnp.int32)
        dst_v[pl.ds(NE, L)] = jnp.full((L,), -1, jnp.int32)
        ea_v[pl.ds(NE, L)] = jnp.zeros((L,), jnp.float32)

        def cbody(c, cnt):
            dv = dst_v[pl.ds(c * L, L)]
            m = (dv >= lo) & (dv < lo + RPT)
            mi = m.astype(jnp.int32)
            incl = plsc.cumsum(mi)
            pos = cnt + incl - mi
            plsc.store_scatter(own_v, [pos], iota + c * L, mask=m)
            return cnt + incl[L - 1]

        cnt = lax.fori_loop(0, NCH, cbody, jnp.int32(0))
        own_v[pl.ds(cnt, L)] = jnp.full((L,), NE, jnp.int32)
        nchunks = lax.shift_right_logical(cnt + (L - 1), 4)

        def abody(c, carry):
            ids = own_v[pl.ds(c * L, L)]
            src_own[pl.ds(c * L, L)] = plsc.load_gather(src_v, [ids])
            dst_own[pl.ds(c * L, L)] = plsc.load_gather(dst_v, [ids])
            ea_own[pl.ds(c * L, L)] = plsc.load_gather(ea_v, [ids])
            return carry

        lax.fori_loop(0, nchunks, abody, 0)
        src_own[pl.ds(NE, L)] = jnp.zeros((L,), jnp.int32)
        dst_own[pl.ds(NE, L)] = jnp.full((L,), -1, jnp.int32)

        def zd(c, carry):
            deg_v[pl.ds(c * L, L)] = jnp.zeros((L,), jnp.int32)
            return carry

        lax.fori_loop(0, NCHO + 1, zd, 0)

        def dcount(j, carry):
            d = dst_own[pl.ds(j, L)][0] - lo
            cdeg = deg_v[pl.ds(d, L)][0]
            plsc.store_scatter(deg_v, [jnp.full((L,), d, jnp.int32)],
                               jnp.full((L,), cdeg + 1, jnp.int32), mask=lane0)
            return carry

        lax.fori_loop(0, cnt, dcount, 0)

        def obody(c, carry_off):
            v = deg_v[pl.ds(c * L, L)]
            incl = plsc.cumsum(v)
            off_v[pl.ds(c * L, L)] = carry_off + incl - v
            cur_v[pl.ds(c * L, L)] = carry_off + incl - v
            return carry_off + incl[L - 1]

        lax.fori_loop(0, NCHO, obody, jnp.int32(0))

        def sfill(j, carry):
            d = dst_own[pl.ds(j, L)][0] - lo
            p = cur_v[pl.ds(d, L)][0]
            plsc.store_scatter(sorted_v, [jnp.full((L,), p, jnp.int32)],
                               jnp.full((L,), j, jnp.int32), mask=lane0)
            plsc.store_scatter(cur_v, [jnp.full((L,), d, jnp.int32)],
                               jnp.full((L,), p + 1, jnp.int32), mask=lane0)
            return carry

        lax.fori_loop(0, cnt, sfill, 0)
        sorted_v[pl.ds(cnt, L)] = jnp.full((L,), NE, jnp.int32)
        sorted_v[pl.ds(cnt + L, L)] = jnp.full((L,), NE, jnp.int32)
        sorted_v[pl.ds(cnt + 2 * L, L)] = jnp.full((L,), NE, jnp.int32)
        sorted_v[pl.ds(cnt + 3 * L, L)] = jnp.full((L,), NE, jnp.int32)

        def gbody(g, carry):
            jpos = sorted_v[pl.ds(g * L, L)]
            sv = plsc.load_gather(src_own, [jpos])
            dv = plsc.load_gather(dst_own, [jpos])
            av = plsc.load_gather(ea_own, [jpos])
            gidx_a[...] = sv
            gidx_b[...] = jnp.maximum(dv, 0)
            cp_a = pltpu.async_copy(xl_hbm.at[gidx_a], rows_a, sem_a)
            cp_b = pltpu.async_copy(xr_hbm.at[gidx_b], rows_b, sem_b)
            cp_a.wait()
            cp_b.wait()
            a_s = [av[t] for t in range(L)]
            for h in range(heads):
                def kb(k, accs):
                    base = h * chan + k * L
                    wek = we_v[pl.ds(base, L)]
                    attk = att_v[pl.ds(base, L)]
                    out = []
                    for t in range(L):
                        v = (rows_a[t, pl.ds(base, L)] + rows_b[t, pl.ds(base, L)]
                             + a_s[t] * wek)
                        z = jnp.where(v >= 0.0, v, 0.2 * v)
                        out.append(accs[t] + z * attk)
                    return tuple(out)

                accs = lax.fori_loop(
                    0, C16, kb,
                    tuple(jnp.zeros((L,), jnp.float32) for _ in range(L)))
                for t in range(L):
                    plsc.store_scatter(
                        lg_v, [jnp.full((L,), h * LGS + g * L + t, jnp.int32)],
                        jnp.full((L,), jnp.sum(accs[t]), jnp.float32),
                        mask=lane0)
            return carry

        lax.fori_loop(0, nchunks, gbody, 0)

        def nbody(i, carry):
            s0 = off_v[pl.ds(i, L)][0]
            s1 = off_v[pl.ds(i + 1, L)][0]
            c0 = lax.shift_right_logical(s0, 4)
            c1 = lax.shift_right_logical(s1 + (L - 1), 4)

            def pa(c, mxs):
                idx = iota + c * L
                m = (idx >= s0) & (idx < s1)
                return tuple(
                    jnp.maximum(mxs[h],
                                jnp.where(m, lg_v[pl.ds(h * LGS + c * L, L)], NEG))
                    for h in range(heads))

            mxs = lax.fori_loop(
                c0, c1, pa,
                tuple(jnp.full((L,), NEG, jnp.float32) for _ in range(heads)))
            amax = [jnp.max(mxs[h]) for h in range(heads)]
            amax = [jnp.where(a < -1e29, 0.0, a) for a in amax]

            def pb(c, dsums):
                idx = iota + c * L
                m = (idx >= s0) & (idx < s1)
                return tuple(
                    dsums[h] + jnp.where(
                        m, jnp.exp(lg_v[pl.ds(h * LGS + c * L, L)] - amax[h]),
                        0.0)
                    for h in range(heads))

            dsums = lax.fori_loop(
                c0, c1, pb,
                tuple(jnp.zeros((L,), jnp.float32) for _ in range(heads)))
            winv = [1.0 / jnp.full((L,), jnp.sum(dsums[h]) + 1e-16, jnp.float32)
                    for h in range(heads)]

            def pw(c, carry2):
                idx = iota + c * L
                m = (idx >= s0) & (idx < s1)
                for h in range(heads):
                    wv = (jnp.exp(lg_v[pl.ds(h * LGS + c * L, L)] - amax[h])
                          * winv[h])
                    plsc.store_scatter(w_v, [h * LGS + idx], wv, mask=m)
                return carry2

            lax.fori_loop(c0, c1, pw, 0)
            return carry

        lax.fori_loop(0, RPT, nbody, 0)

        if pooled:
            def zp(k, carry):
                pool_v[pl.ds(k * L, L)] = jnp.zeros((L,), jnp.float32)
                acc_v[pl.ds(k * L, L)] = jnp.zeros((L,), jnp.float32)
                return carry
            lax.fori_loop(0, D16, zp, 0)

            def flush(nid):
                def fk(k, carry):
                    kb = k * L
                    plsc.addupdate(
                        pool_v.at[pl.ds(kb, L)],
                        jnp.maximum(acc_v[pl.ds(kb, L)] + b_v[pl.ds(kb, L)],
                                    0.0))
                    acc_v[pl.ds(kb, L)] = jnp.zeros((L,), jnp.float32)
                    return carry
                lax.fori_loop(0, D16, fk, 0)

            def process(g, cur):
                jpos = sorted_v[pl.ds(g * L, L)]
                srcs = plsc.load_gather(src_own, [jpos])
                dstv = plsc.load_gather(dst_own, [jpos])
                gidx_a[...] = srcs
                pltpu.async_copy(xl_hbm.at[gidx_a], rows_a, sem_a).wait()
                rows = rows_a
                for t in range(L):
                    d_t = dstv[t]

                    @pl.when((d_t >= 0) & (d_t != cur) & (cur >= 0))
                    def _():
                        flush(cur)

                    cur = jnp.where(d_t >= 0, d_t, cur)

                    @pl.when(d_t >= 0)
                    def _():
                        w_t = w_v[pl.ds(g * L + t, L)][0]

                        def kb2(k, carry2):
                            kb = k * L
                            plsc.addupdate(acc_v.at[pl.ds(kb, L)],
                                           w_t * rows[t, pl.ds(kb, L)])
                            return carry2

                        lax.fori_loop(0, D16, kb2, 0)
                return cur

            cur = lax.fori_loop(0, nchunks, process, jnp.int32(-1))

            @pl.when(cur >= 0)
            def _():
                flush(cur)

            pltpu.sync_copy(pool_v, out_hbm.at[wid])
        else:
            def zo(i, carry):
                def zk(k, carry2):
                    outbuf_v[i, pl.ds(k * L, L)] = jnp.zeros((L,), jnp.float32)
                    return carry2
                lax.fori_loop(0, D16, zk, 0)
                return carry

            lax.fori_loop(0, RPT, zo, 0)

            def process(g, carry):
                jpos = sorted_v[pl.ds(g * L, L)]
                srcs = plsc.load_gather(src_own, [jpos])
                dstv = plsc.load_gather(dst_own, [jpos])
                gidx_a[...] = srcs
                pltpu.async_copy(xl_hbm.at[gidx_a], rows_a, sem_a).wait()
                rows = rows_a
                for t in range(L):
                    d_t = dstv[t]

                    @pl.when(d_t >= 0)
                    def _():
                        dl = d_t - lo
                        for h in range(heads):
                            w_t = w_v[pl.ds(h * LGS + g * L + t, L)][0]

                            def kb2(k, carry2):
                                base = h * chan + k * L
                                plsc.addupdate(
                                    outbuf_v.at[dl, pl.ds(base, L)],
                                    w_t * rows[t, pl.ds(base, L)])
                                return carry2

                            lax.fori_loop(0, C16, kb2, 0)
                return carry

            lax.fori_loop(0, nchunks, process, 0)
            pltpu.sync_copy(outbuf_v, out_hbm.at[pl.ds(lo, RPT)])

    return body(xl, xr, src, dst, eav, wev, attv,
                bias if bias is not None else jnp.zeros((D,), jnp.float32))


def kernel(x, edge_index, edge_attr, g1_Wl, g1_bl, g1_Wr, g1_br, g1_We, g1_att,
           g1_b, g2_Wl, g2_bl, g2_Wr, g2_br, g2_We, g2_att, g2_b, alpha):
    N, F = x.shape
    E = edge_index.shape[1]
    NE = E + N
    H1, C1 = g1_att.shape
    D1 = H1 * C1
    A = E + 1
    D2 = -(-A // 128) * 128

    loop = jnp.arange(N, dtype=edge_index.dtype)
    src = jnp.concatenate([edge_index[0], loop])
    dst = jnp.concatenate([edge_index[1], loop])

    xl1, xr1, mean_ea = _tc_pre(x, g1_Wl, g1_bl, g1_Wr, g1_br, edge_attr)
    eav = jnp.concatenate(
        [edge_attr[:, 0], jnp.broadcast_to(mean_ea[0, 0], (N,))])

    h_pre = _sc_layer(
        xl1, xr1, src, dst, eav, g1_We[0], g1_att.reshape(D1),
        heads=H1, chan=C1, n_nodes=N, pooled=False)

    pad = D2 - A
    W2l = jnp.pad(g2_Wl, ((0, 0), (0, pad)))
    W2r = jnp.pad(g2_Wr, ((0, 0), (0, pad)))
    b2l = jnp.pad(g2_bl, (0, pad))
    b2r = jnp.pad(g2_br, (0, pad))
    we2 = jnp.pad(g2_We[0], (0, pad))
    att2 = jnp.pad(g2_att.reshape(A), (0, pad))
    b2 = jnp.pad(g2_b, (0, pad))

    xl2, xr2 = _tc_mid(h_pre[:N], g1_b, W2l, b2l, W2r, b2r)

    pool_parts = _sc_layer(
        xl2, xr2, src, dst, eav, we2, att2,
        heads=1, chan=D2, n_nodes=N, pooled=True, bias=b2)

    probs_full, sig11 = _tc_post(pool_parts, alpha.reshape(1, 1), N, A)
    return probs_full[:, :A], sig11.reshape(())
